# baseline (speedup 1.0000x reference)
import functools
import math

import jax
import jax.numpy as jnp
from jax import lax
from jax.experimental import pallas as pl
from jax.experimental.pallas import tpu as pltpu
from jax.experimental.pallas import tpu_sc as plsc

EMB = 32
NCORES = 2
NSUB = 16
NW = NCORES * NSUB
SCALE = math.sqrt(EMB)

SUB = 128
CHUNK = 640
K = CHUNK // SUB
NBUF = 5
LEAD = 2


@functools.lru_cache(maxsize=None)
def _build_prescale(V):
    BLK = 4096
    grid = (V + BLK - 1) // BLK

    def tbody(wt_ref, out_ref):
        i = jax.lax.broadcasted_iota(jnp.int32, (EMB, EMB), 0)
        j = jax.lax.broadcasted_iota(jnp.int32, (EMB, EMB), 1)
        eye_s = jnp.where(i == j, jnp.float32(SCALE), jnp.float32(0.0))
        out_ref[...] = jax.lax.dot_general(
            wt_ref[...], eye_s, (((0,), (0,)), ((), ())),
            preferred_element_type=jnp.float32)

    return pl.pallas_call(
        tbody,
        grid=(grid,),
        in_specs=[pl.BlockSpec((EMB, BLK), lambda i: (0, i))],
        out_specs=pl.BlockSpec((BLK, EMB), lambda i: (i, 0)),
        out_shape=jax.ShapeDtypeStruct((V, EMB), jnp.float32),
    )


@functools.lru_cache(maxsize=None)
def _build(B0, H, V):
    rows_w = B0 * H // NW
    bats_w = rows_w // H
    iters = rows_w // CHUNK
    outer = iters // NBUF
    bats_g = CHUNK * NBUF // H
    assert rows_w % CHUNK == 0 and iters % NBUF == 0 and LEAD < NBUF
    assert (CHUNK * NBUF) % H == 0
    SPLITS = []
    for s in range(NBUF):
        start = s * CHUNK
        pieces = []
        o = 0
        h = start % H
        db = start // H
        while o < CHUNK:
            n = min(H - h, CHUNK - o)
            pieces.append((o, n, db, h))
            o += n
            h = 0
            db += 1
        SPLITS.append(pieces)

    mesh = plsc.VectorSubcoreMesh(
        core_axis_name="c", subcore_axis_name="s",
        num_cores=NCORES, num_subcores=NSUB)

    def body(w_hbm, idx_hbm, out_hbm, idx_v, rows_v,
             g0, g1, g2, g3, g4, o0, o1, o2, o3, o4):
        gsem = [g0, g1, g2, g3, g4]
        osem = [o0, o1, o2, o3, o4]
        wid = lax.axis_index("s") * NCORES + lax.axis_index("c")
        sub_base = wid * (rows_w // SUB)
        bat_base = wid * bats_w

        def gather_descs(b):
            return [
                pltpu.make_async_copy(
                    w_hbm.at[idx_v.at[b, j]],
                    rows_v.at[b, pl.ds(j * SUB, SUB)],
                    gsem[b])
                for j in range(K)
            ]

        def fire(c, b):
            pltpu.sync_copy(idx_hbm.at[pl.ds(sub_base + c * K, K)],
                            idx_v.at[b])
            for d in gather_descs(b):
                d.start()

        def out_descs(g, s):
            b0 = bat_base + g * bats_g
            return [
                pltpu.make_async_copy(
                    rows_v.at[s, pl.ds(o, n)],
                    out_hbm.at[b0 + db, pl.ds(h0, n)],
                    osem[s])
                for (o, n, db, h0) in SPLITS[s]
            ]

        for b in range(LEAD):
            fire(b, b)

        def step(g, carry):
            for s in range(NBUF):
                c = g * NBUF + s
                b = s
                for d in gather_descs(b):
                    d.wait()
                for d in out_descs(g, s):
                    d.start()
                c2 = c + LEAD
                s2 = (s + LEAD) % NBUF
                sp = (s + LEAD - NBUF) % NBUF
                gp = g + (s + LEAD - NBUF) // NBUF

                @pl.when(c2 >= NBUF)
                def _():
                    for d in out_descs(gp, sp):
                        d.wait()

                @pl.when(c2 < iters)
                def _():
                    fire(c2, s2)
            return carry

        lax.fori_loop(0, outer, step, 0)

        for s in range(LEAD, NBUF):
            for d in out_descs(outer - 1, s):
                d.wait()

    return pl.kernel(
        body,
        out_type=jax.ShapeDtypeStruct((B0, H, EMB), jnp.float32),
        mesh=mesh,
        compiler_params=pltpu.CompilerParams(use_tc_tiling_on_sc=False),
        scratch_types=[
            pltpu.VMEM((NBUF, K, SUB), jnp.int32),
            pltpu.VMEM((NBUF, CHUNK, EMB), jnp.float32),
        ] + [pltpu.SemaphoreType.DMA] * (2 * NBUF),
    )


def kernel(x, weight):
    B0, H = x.shape
    V = weight.shape[0]
    idx2d = x.reshape(B0 * H // SUB, SUB).astype(jnp.int32)
    wlin = _build_prescale(V)(weight.T)
    return _build(B0, H, V)(wlin, idx2d)


# speedup vs baseline: 1.1050x; 1.1050x over previous
import functools
import math

import jax
import jax.numpy as jnp
from jax import lax
from jax.experimental import pallas as pl
from jax.experimental.pallas import tpu as pltpu
from jax.experimental.pallas import tpu_sc as plsc

EMB = 32
NCORES = 2
NSUB = 16
NW = NCORES * NSUB
SCALE = math.sqrt(EMB)

SUB = 128
CHUNK = 640
K = CHUNK // SUB
NBUF = 5
LEAD = 2


@functools.lru_cache(maxsize=None)
def _build_prescale(V):
    BLK = 16384
    grid = (V + BLK - 1) // BLK

    def tbody(wt_ref, out_ref):
        i = jax.lax.broadcasted_iota(jnp.int32, (EMB, EMB), 0)
        j = jax.lax.broadcasted_iota(jnp.int32, (EMB, EMB), 1)
        eye_s = jnp.where(i == j, jnp.float32(SCALE), jnp.float32(0.0))
        out_ref[...] = jax.lax.dot_general(
            wt_ref[...], eye_s, (((0,), (0,)), ((), ())),
            preferred_element_type=jnp.float32)

    return pl.pallas_call(
        tbody,
        grid=(grid,),
        in_specs=[pl.BlockSpec((EMB, BLK), lambda i: (0, i))],
        out_specs=pl.BlockSpec((BLK, EMB), lambda i: (i, 0)),
        out_shape=jax.ShapeDtypeStruct((V, EMB), jnp.float32),
    )


@functools.lru_cache(maxsize=None)
def _build(B0, H, V):
    rows_w = B0 * H // NW
    bats_w = rows_w // H
    iters = rows_w // CHUNK
    outer = iters // NBUF
    bats_g = CHUNK * NBUF // H
    assert rows_w % CHUNK == 0 and iters % NBUF == 0 and LEAD < NBUF
    assert (CHUNK * NBUF) % H == 0
    SPLITS = []
    for s in range(NBUF):
        start = s * CHUNK
        pieces = []
        o = 0
        h = start % H
        db = start // H
        while o < CHUNK:
            n = min(H - h, CHUNK - o)
            pieces.append((o, n, db, h))
            o += n
            h = 0
            db += 1
        SPLITS.append(pieces)

    mesh = plsc.VectorSubcoreMesh(
        core_axis_name="c", subcore_axis_name="s",
        num_cores=NCORES, num_subcores=NSUB)

    def body(w_hbm, idx_hbm, out_hbm, idx_v, rows_v,
             g0, g1, g2, g3, g4, o0, o1, o2, o3, o4):
        gsem = [g0, g1, g2, g3, g4]
        osem = [o0, o1, o2, o3, o4]
        wid = lax.axis_index("s") * NCORES + lax.axis_index("c")
        sub_base = wid * (rows_w // SUB)
        bat_base = wid * bats_w

        def gather_descs(b):
            return [
                pltpu.make_async_copy(
                    w_hbm.at[idx_v.at[b, j]],
                    rows_v.at[b, pl.ds(j * SUB, SUB)],
                    gsem[b])
                for j in range(K)
            ]

        def fire(c, b):
            pltpu.sync_copy(idx_hbm.at[pl.ds(sub_base + c * K, K)],
                            idx_v.at[b])
            for d in gather_descs(b):
                d.start()

        def out_descs(g, s):
            b0 = bat_base + g * bats_g
            return [
                pltpu.make_async_copy(
                    rows_v.at[s, pl.ds(o, n)],
                    out_hbm.at[b0 + db, pl.ds(h0, n)],
                    osem[s])
                for (o, n, db, h0) in SPLITS[s]
            ]

        for b in range(LEAD):
            fire(b, b)

        def step(g, carry):
            for s in range(NBUF):
                c = g * NBUF + s
                b = s
                for d in gather_descs(b):
                    d.wait()
                for d in out_descs(g, s):
                    d.start()
                c2 = c + LEAD
                s2 = (s + LEAD) % NBUF
                sp = (s + LEAD - NBUF) % NBUF
                gp = g + (s + LEAD - NBUF) // NBUF

                @pl.when(c2 >= NBUF)
                def _():
                    for d in out_descs(gp, sp):
                        d.wait()

                @pl.when(c2 < iters)
                def _():
                    fire(c2, s2)
            return carry

        lax.fori_loop(0, outer, step, 0)

        for s in range(LEAD, NBUF):
            for d in out_descs(outer - 1, s):
                d.wait()

    return pl.kernel(
        body,
        out_type=jax.ShapeDtypeStruct((B0, H, EMB), jnp.float32),
        mesh=mesh,
        compiler_params=pltpu.CompilerParams(use_tc_tiling_on_sc=False),
        scratch_types=[
            pltpu.VMEM((NBUF, K, SUB), jnp.int32),
            pltpu.VMEM((NBUF, CHUNK, EMB), jnp.float32),
        ] + [pltpu.SemaphoreType.DMA] * (2 * NBUF),
    )


def kernel(x, weight):
    B0, H = x.shape
    V = weight.shape[0]
    idx2d = x.reshape(B0 * H // SUB, SUB).astype(jnp.int32)
    wlin = _build_prescale(V)(weight.T)
    return _build(B0, H, V)(wlin, idx2d)


# speedup vs baseline: 1.1649x; 1.0543x over previous
import functools
import math

import jax
import jax.numpy as jnp
from jax import lax
from jax.experimental import pallas as pl
from jax.experimental.pallas import tpu as pltpu
from jax.experimental.pallas import tpu_sc as plsc

EMB = 32
NCORES = 2
NSUB = 16
NW = NCORES * NSUB
SCALE = math.sqrt(EMB)

SUB = 128
CHUNK = 640
K = CHUNK // SUB
NBUF = 5
LEAD = 2


@functools.lru_cache(maxsize=None)
def _build(B0, H, V):
    rows_w = B0 * H // NW
    bats_w = rows_w // H
    iters = rows_w // CHUNK
    outer = iters // NBUF
    bats_g = CHUNK * NBUF // H
    assert rows_w % CHUNK == 0 and iters % NBUF == 0 and LEAD < NBUF
    assert (CHUNK * NBUF) % H == 0
    SPLITS = []
    for s in range(NBUF):
        start = s * CHUNK
        pieces = []
        o = 0
        h = start % H
        db = start // H
        while o < CHUNK:
            n = min(H - h, CHUNK - o)
            pieces.append((o, n, db, h))
            o += n
            h = 0
            db += 1
        SPLITS.append(pieces)

    mesh = plsc.VectorSubcoreMesh(
        core_axis_name="c", subcore_axis_name="s",
        num_cores=NCORES, num_subcores=NSUB)

    def body(w_hbm, idx_hbm, out_hbm, idx_v, rows_v,
             g0, g1, g2, g3, g4, o0, o1, o2, o3, o4):
        gsem = [g0, g1, g2, g3, g4]
        osem = [o0, o1, o2, o3, o4]
        wid = lax.axis_index("s") * NCORES + lax.axis_index("c")
        sub_base = wid * (rows_w // SUB)
        bat_base = wid * bats_w

        def gather_descs(b):
            return [
                pltpu.make_async_copy(
                    w_hbm.at[idx_v.at[b, j]],
                    rows_v.at[b, pl.ds(j * SUB, SUB)],
                    gsem[b])
                for j in range(K)
            ]

        def fire(c, b):
            pltpu.sync_copy(idx_hbm.at[pl.ds(sub_base + c * K, K)],
                            idx_v.at[b])
            for d in gather_descs(b):
                d.start()

        def out_descs(g, s):
            b0 = bat_base + g * bats_g
            return [
                pltpu.make_async_copy(
                    rows_v.at[s, pl.ds(o, n)],
                    out_hbm.at[b0 + db, pl.ds(h0, n)],
                    osem[s])
                for (o, n, db, h0) in SPLITS[s]
            ]

        def scale(b):
            def scale_step(i, carry):
                base = i * 8
                for r in range(8):
                    for h in range(2):
                        sl = pl.ds(h * 16, 16)
                        rows_v[b, base + r, sl] = rows_v[b, base + r, sl] * SCALE
                return carry

            lax.fori_loop(0, CHUNK // 8, scale_step, 0)

        for b in range(LEAD):
            fire(b, b)

        def step(g, carry):
            for s in range(NBUF):
                c = g * NBUF + s
                b = s
                for d in gather_descs(b):
                    d.wait()
                scale(b)
                for d in out_descs(g, s):
                    d.start()
                c2 = c + LEAD
                s2 = (s + LEAD) % NBUF
                sp = (s + LEAD - NBUF) % NBUF
                gp = g + (s + LEAD - NBUF) // NBUF

                @pl.when(c2 >= NBUF)
                def _():
                    for d in out_descs(gp, sp):
                        d.wait()

                @pl.when(c2 < iters)
                def _():
                    fire(c2, s2)
            return carry

        lax.fori_loop(0, outer, step, 0)

        for s in range(LEAD, NBUF):
            for d in out_descs(outer - 1, s):
                d.wait()

    return pl.kernel(
        body,
        out_type=jax.ShapeDtypeStruct((B0, H, EMB), jnp.float32),
        mesh=mesh,
        compiler_params=pltpu.CompilerParams(use_tc_tiling_on_sc=False),
        scratch_types=[
            pltpu.VMEM((NBUF, K, SUB), jnp.int32),
            pltpu.VMEM((NBUF, CHUNK, EMB), jnp.float32),
        ] + [pltpu.SemaphoreType.DMA] * (2 * NBUF),
    )


def kernel(x, weight):
    B0, H = x.shape
    idx2d = x.reshape(B0 * H // SUB, SUB).astype(jnp.int32)
    return _build(B0, H, weight.shape[0])(weight, idx2d)
